# baseline (speedup 1.0000x reference)
import functools

import jax
import jax.numpy as jnp
from jax import lax
from jax.experimental import pallas as pl
from jax.experimental.pallas import tpu as pltpu
from jax.experimental.pallas import tpu_sc as plsc

NC = 2
NS = 16
NW = NC * NS
CH = 128
LANES = 16


def _make_body(total, S, D):
    per_w = total // NW
    n_chunks = per_w // CH
    n_col = D // LANES

    def body(x_hbm, posdup_hbm, tok_hbm, out_hbm, pos_v, idx_v, rows_v, sem):
        wid = lax.axis_index("s") * NC + lax.axis_index("c")
        base0 = wid * per_w
        pltpu.sync_copy(posdup_hbm, pos_v)

        def chunk_body(k, carry):
            base = base0 + k * CH
            pltpu.sync_copy(x_hbm.at[pl.ds(base, CH)], idx_v)
            pltpu.async_copy(tok_hbm.at[idx_v], rows_v, sem).wait()
            p0 = lax.rem(base, S)

            def row_body(r, c2):
                for c in range(n_col):
                    sl = pl.ds(c * LANES, LANES)
                    plsc.addupdate(rows_v.at[r, sl], pos_v[p0 + r, sl])
                return c2

            lax.fori_loop(0, CH, row_body, 0)
            pltpu.sync_copy(rows_v, out_hbm.at[pl.ds(base, CH)])
            return carry

        lax.fori_loop(0, n_chunks, chunk_body, 0)

    return body


@functools.partial(jax.jit, static_argnames=())
def kernel(x, tok_table, pos_table):
    B, S = x.shape
    V, D = tok_table.shape
    total = B * S
    xf = x.reshape(total).astype(jnp.int32)
    posdup = jnp.concatenate([pos_table, pos_table], axis=0)

    mesh = plsc.VectorSubcoreMesh(core_axis_name="c", subcore_axis_name="s")
    run = pl.kernel(
        _make_body(total, S, D),
        mesh=mesh,
        compiler_params=pltpu.CompilerParams(use_tc_tiling_on_sc=False),
        out_type=jax.ShapeDtypeStruct((total, D), jnp.float32),
        scratch_types=[
            pltpu.VMEM((2 * S, D), jnp.float32),
            pltpu.VMEM((CH,), jnp.int32),
            pltpu.VMEM((CH, D), jnp.float32),
            pltpu.SemaphoreType.DMA,
        ],
    )
    out = run(xf, posdup, tok_table)
    return out.reshape(B, S, D)


# speedup vs baseline: 1.0103x; 1.0103x over previous
import functools

import jax
import jax.numpy as jnp
from jax import lax
from jax.experimental import pallas as pl
from jax.experimental.pallas import tpu as pltpu
from jax.experimental.pallas import tpu_sc as plsc

NC = 2
NS = 16
NW = NC * NS
CH = 128
LANES = 16


def _make_body(total, S, D):
    per_w = total // NW
    n_chunks = per_w // CH
    n_col = D // LANES

    def body(x_hbm, tok2_hbm, posdup_hbm, out_hbm,
             pos_v, idx_v, g_v, h_v, rows2_v, out_v, sem):
        wid = lax.axis_index("s") * NC + lax.axis_index("c")
        base0 = wid * per_w
        pltpu.sync_copy(posdup_hbm, pos_v)
        iotas = [lax.iota(jnp.int32, LANES) + c * LANES for c in range(n_col)]

        def chunk_body(k, carry):
            base = base0 + k * CH
            pltpu.sync_copy(x_hbm.at[pl.ds(base, CH)], idx_v)
            for j in range(CH // LANES):
                sl = pl.ds(j * LANES, LANES)
                v = idx_v[sl]
                g_v[sl] = lax.shift_right_logical(v, 1)
                h_v[sl] = lax.shift_left((v & 1), 6)
            pltpu.async_copy(tok2_hbm.at[g_v], rows2_v, sem).wait()
            p0 = lax.rem(base, S)

            def row_body(j, c2):
                hvec = h_v[pl.ds(j * LANES, LANES)]
                for i in range(LANES):
                    r = j * LANES + i
                    off = hvec[i]
                    for c in range(n_col):
                        sl = pl.ds(c * LANES, LANES)
                        out_v[r, sl] = (rows2_v[r, pl.ds(off + c * LANES, LANES)]
                                        + pos_v[p0 + r, sl])
                return c2

            lax.fori_loop(0, CH // LANES, row_body, 0)

            pltpu.sync_copy(out_v, out_hbm.at[pl.ds(base, CH)])
            return carry

        lax.fori_loop(0, n_chunks, chunk_body, 0)

    return body


@functools.partial(jax.jit, static_argnames=())
def kernel(x, tok_table, pos_table):
    B, S = x.shape
    V, D = tok_table.shape
    total = B * S
    xf = x.reshape(total).astype(jnp.int32)
    tok2 = tok_table.reshape(V // 2, 2 * D)
    posdup = jnp.concatenate([pos_table, pos_table], axis=0)

    mesh = plsc.VectorSubcoreMesh(core_axis_name="c", subcore_axis_name="s")
    run = pl.kernel(
        _make_body(total, S, D),
        mesh=mesh,
        out_type=jax.ShapeDtypeStruct((total, D), jnp.float32),
        scratch_types=[
            pltpu.VMEM((2 * S, D), jnp.float32),
            pltpu.VMEM((CH,), jnp.int32),
            pltpu.VMEM((CH,), jnp.int32),
            pltpu.VMEM((CH,), jnp.int32),
            pltpu.VMEM((CH, 2 * D), jnp.float32),
            pltpu.VMEM((CH, D), jnp.float32),
            pltpu.SemaphoreType.DMA,
        ],
    )
    out = run(xf, tok2, posdup)
    return out.reshape(B, S, D)


# speedup vs baseline: 1.0470x; 1.0363x over previous
import functools

import jax
import jax.numpy as jnp
from jax import lax
from jax.experimental import pallas as pl
from jax.experimental.pallas import tpu as pltpu
from jax.experimental.pallas import tpu_sc as plsc

NC = 2
NS = 16
NW = NC * NS
CH = 128
LANES = 16
NBUF = 2


def _make_body(total, S, D):
    per_w = total // NW
    n_chunks = per_w // CH
    n_col = D // LANES
    assert n_chunks >= 2 * NBUF and (n_chunks - 2 * NBUF) % NBUF == 0

    def body(x_hbm, posdup_hbm, tok_hbm, out_hbm,
             pos_v, idx_v, rows_v, out_v, gsems, osems):
        wid = lax.axis_index("s") * NC + lax.axis_index("c")
        base0 = wid * per_w
        pltpu.sync_copy(posdup_hbm, pos_v)

        def gather(k, slot):
            base = base0 + k * CH
            return pltpu.make_async_copy(
                tok_hbm.at[idx_v.at[slot]], rows_v.at[slot], gsems.at[slot])

        def store(k, slot):
            base = base0 + k * CH
            return pltpu.make_async_copy(
                out_v.at[slot], out_hbm.at[pl.ds(base, CH)], osems.at[slot])

        def start_chunk(k, slot):
            base = base0 + k * CH
            pltpu.sync_copy(x_hbm.at[pl.ds(base, CH)], idx_v.at[slot])
            gather(k, slot).start()

        def compute_chunk(k, slot):
            p0 = lax.rem(base0 + k * CH, S)

            def row_body(j, c2):
                r0 = j * LANES
                for i in range(LANES):
                    r = r0 + i
                    pr = p0 + r
                    for c in range(n_col):
                        sl = pl.ds(c * LANES, LANES)
                        out_v[slot, r, sl] = rows_v[slot, r, sl] + pos_v[pr, sl]
                return c2

            lax.fori_loop(0, CH // LANES, row_body, 0)

        for s in range(NBUF):
            start_chunk(s, s)
        for k in range(NBUF):
            gather(k, k).wait()
            compute_chunk(k, k)
            store(k, k).start()
            start_chunk(k + NBUF, k)

        def main_body(k2, c2):
            for b in range(NBUF):
                k = NBUF + k2 * NBUF + b
                gather(k, b).wait()
                store(k - NBUF, b).wait()
                compute_chunk(k, b)
                store(k, b).start()
                start_chunk(k + NBUF, b)
            return c2

        lax.fori_loop(0, (n_chunks - 2 * NBUF) // NBUF, main_body, 0)

        for k in range(n_chunks - NBUF, n_chunks):
            slot = k % NBUF
            gather(k, slot).wait()
            store(k - NBUF, slot).wait()
            compute_chunk(k, slot)
            store(k, slot).start()
        for k in range(n_chunks - NBUF, n_chunks):
            store(k, k % NBUF).wait()

    return body


@functools.partial(jax.jit, static_argnames=())
def kernel(x, tok_table, pos_table):
    B, S = x.shape
    V, D = tok_table.shape
    total = B * S
    xf = x.reshape(total).astype(jnp.int32)
    posdup = jnp.concatenate([pos_table, pos_table], axis=0)

    mesh = plsc.VectorSubcoreMesh(core_axis_name="c", subcore_axis_name="s")
    run = pl.kernel(
        _make_body(total, S, D),
        mesh=mesh,
        compiler_params=pltpu.CompilerParams(use_tc_tiling_on_sc=False),
        out_type=jax.ShapeDtypeStruct((total, D), jnp.float32),
        scratch_types=[
            pltpu.VMEM((2 * S, D), jnp.float32),
            pltpu.VMEM((NBUF, CH), jnp.int32),
            pltpu.VMEM((NBUF, CH, D), jnp.float32),
            pltpu.VMEM((NBUF, CH, D), jnp.float32),
            pltpu.SemaphoreType.DMA((NBUF,)),
            pltpu.SemaphoreType.DMA((NBUF,)),
        ],
    )
    out = run(xf, posdup, tok_table)
    return out.reshape(B, S, D)
